# Initial kernel scaffold; baseline (speedup 1.0000x reference)
import jax
import jax.numpy as jnp
from jax.experimental import pallas as pl


def kernel(x, embed_weight):
    raise NotImplementedError("write your pallas kernel here")


# speedup vs baseline: 7.8937x; 7.8937x over previous
import functools
import math

import jax
import jax.numpy as jnp
from jax import lax
from jax.experimental import pallas as pl
from jax.experimental.pallas import tpu as pltpu
from jax.experimental.pallas import tpu_sc as plsc

D_MODEL = 128
B_TOTAL = 1024 * 200
NW = 32
PER_W = B_TOTAL // NW
K = 128
NCH = PER_W // K
NBUF = 5
NGRP = NCH // NBUF
SCALE = float(math.sqrt(D_MODEL))

_mesh = plsc.VectorSubcoreMesh(core_axis_name="c", subcore_axis_name="s")


def _scale_rows(rows_ref):
    def body(i, carry):
        for j in range(D_MODEL // 16):
            sl = pl.ds(j * 16, 16)
            rows_ref[i, sl] = rows_ref[i, sl] * SCALE
        return carry
    lax.fori_loop(0, K, body, 0, unroll=2)


@functools.partial(
    pl.kernel,
    mesh=_mesh,
    out_type=jax.ShapeDtypeStruct((B_TOTAL, D_MODEL), jnp.float32),
    scratch_types=(
        [pltpu.VMEM((NCH, K), jnp.int32)]
        + [pltpu.VMEM((K, D_MODEL), jnp.float32) for _ in range(NBUF)]
        + [pltpu.SemaphoreType.DMA for _ in range(2 * NBUF)]
    ),
)
def _emb_lookup(x_hbm, w_hbm, out_hbm, idx_v, *bufs_and_sems):
    rows = list(bufs_and_sems[:NBUF])
    gsem = list(bufs_and_sems[NBUF:2 * NBUF])
    ssem = list(bufs_and_sems[2 * NBUF:])

    wid = lax.axis_index("s") * 2 + lax.axis_index("c")
    base_row = wid * PER_W

    pltpu.sync_copy(x_hbm.at[wid], idx_v)

    def issue_gather(c, b):
        pltpu.async_copy(w_hbm.at[idx_v.at[c]], rows[b], gsem[b])

    def wait_gather(b):
        pltpu.make_async_copy(w_hbm.at[idx_v.at[0]], rows[b], gsem[b]).wait()

    def issue_store(c, b):
        pltpu.async_copy(rows[b], out_hbm.at[pl.ds(base_row + c * K, K)], ssem[b])

    def wait_store(b):
        pltpu.make_async_copy(
            rows[b], out_hbm.at[pl.ds(base_row, K)], ssem[b]
        ).wait()

    for b in range(NBUF):
        issue_gather(b, b)

    def do_chunk(g, b, reuse):
        if reuse:
            pb = (b - 1) % NBUF
            wait_store(pb)
            issue_gather(g - 1 + NBUF, pb)
        wait_gather(b)
        _scale_rows(rows[b])
        issue_store(g, b)

    do_chunk(0, 0, reuse=False)
    for b in range(1, NBUF):
        do_chunk(b, b, reuse=True)

    def group(o, carry):
        for b in range(NBUF):
            g = o * NBUF + b
            pb = (b - 1) % NBUF
            wait_store(pb)
            pltpu.async_copy(w_hbm.at[idx_v.at[g - 1 + NBUF]], rows[pb], gsem[pb])
            wait_gather(b)
            _scale_rows(rows[b])
            pltpu.async_copy(
                rows[b], out_hbm.at[pl.ds(base_row + g * K, K)], ssem[b]
            )
        return carry
    lax.fori_loop(1, NGRP - 1, group, 0)

    g0 = (NGRP - 1) * NBUF
    do_chunk(g0, 0, reuse=True)
    for b in range(1, NBUF):
        do_chunk(g0 + b, b, reuse=False)

    for b in range(NBUF):
        wait_store(b)


def kernel(x, embed_weight):
    xf = x.reshape(-1).astype(jnp.int32).reshape(NW, NCH, K)
    out = _emb_lookup(xf, embed_weight)
    return out.reshape(x.shape[0], x.shape[1], D_MODEL)
